# baseline (speedup 1.0000x reference)
import functools

import jax
import jax.numpy as jnp
from jax import lax
from jax.experimental import pallas as pl
from jax.experimental.pallas import tpu as pltpu
from jax.experimental.pallas import tpu_sc as plsc

STRENGTH = 0.9
THRESHOLD = 0.5

ROWS, COLS = 16384, 1024
NC, NS, L = 2, 16, 16
NW = NC * NS
SC_ROWS = 2048
RPW = SC_ROWS // NW
TC_BR = 2048

_MESH = plsc.VectorSubcoreMesh(
    core_axis_name="c", subcore_axis_name="s", num_cores=NC, num_subcores=NS
)


@functools.partial(
    pl.kernel,
    out_type=jax.ShapeDtypeStruct((SC_ROWS, COLS), jnp.float32),
    mesh=_MESH,
    scratch_types=[
        pltpu.VMEM((RPW, COLS), jnp.float32),
        pltpu.SemaphoreType.DMA,
    ],
    compiler_params=pltpu.CompilerParams(needs_layout_passes=False),
)
def _sc_tile(z_hbm, t_hbm, buf, sem):
    wid = lax.axis_index("s") * NC + lax.axis_index("c")
    base = wid * RPW
    pltpu.async_copy(z_hbm.at[pl.ds(base, RPW)], buf, sem).wait()
    rid = jnp.arange(L, dtype=jnp.int32)
    zero16 = jnp.zeros((L,), jnp.int32)
    for h in range(RPW // L):
        r = rid + h * L
        vals = plsc.load_gather(buf, [r, zero16])
        wet = STRENGTH / (1.0 + jnp.exp((THRESHOLD - vals) * 10.0))
        plsc.store_scatter(buf, [r, zero16 + 1], wet)
    pltpu.sync_copy(buf, t_hbm.at[pl.ds(base, RPW)])


def _bulk_body(z_ref, o_ref):
    zb = z_ref[...]
    wet = jax.nn.sigmoid((zb[:, 0:1] - THRESHOLD) * 10.0) * STRENGTH
    lane = lax.broadcasted_iota(jnp.int32, zb.shape, 1)
    o_ref[...] = jnp.where(lane == 1, wet, zb)


def _tc_bulk(z):
    grid = ((ROWS - SC_ROWS) // TC_BR,)
    return pl.pallas_call(
        _bulk_body,
        grid=grid,
        in_specs=[pl.BlockSpec((TC_BR, COLS), lambda i: (i + 1, 0))],
        out_specs=pl.BlockSpec((TC_BR, COLS), lambda i: (i + 1, 0)),
        out_shape=jax.ShapeDtypeStruct((ROWS, COLS), jnp.float32),
    )(z)


def _merge_body(t_ref, a_ref, o_ref):
    o_ref[...] = t_ref[...]


def _tc_merge(t, a):
    return pl.pallas_call(
        _merge_body,
        grid=(1,),
        in_specs=[
            pl.BlockSpec((SC_ROWS, COLS), lambda i: (0, 0)),
            pl.BlockSpec(memory_space=pltpu.MemorySpace.HBM),
        ],
        out_specs=pl.BlockSpec((SC_ROWS, COLS), lambda i: (0, 0)),
        out_shape=jax.ShapeDtypeStruct((ROWS, COLS), jnp.float32),
        input_output_aliases={1: 0},
    )(t, a)


def kernel(z):
    t = _sc_tile(z)
    a = _tc_bulk(z)
    return _tc_merge(t, a)
